# Initial kernel scaffold; baseline (speedup 1.0000x reference)
import jax
import jax.numpy as jnp
from jax.experimental import pallas as pl


def kernel(data, indices_identity, indices_transpose, data_embedding, data_target, W_in, b_in, Wpr0, Wpc0, Wpt0, bp0, Wbr0, Wbc0, bb0, Wpr1, Wpc1, Wpt1, bp1, Wbr1, Wbc1, bb1, W_out, b_out):
    raise NotImplementedError("write your pallas kernel here")


# speedup vs baseline: 4.1342x; 4.1342x over previous
import functools

import jax
import jax.numpy as jnp
from jax import lax
from jax.experimental import pallas as pl
from jax.experimental.pallas import tpu as pltpu
from jax.experimental.pallas import tpu_sc as plsc

N = 50000
E = 800000
ET = 200000
CIN = 16
WIDTH = 32
EMB = 64
EPS = 1e-5
F32 = jnp.float32

NC = 2
NS = 16
NW = NC * NS
NPAD = 50176
RPT = NPAD // NS
CH = 128
EP2 = 819200
EPT = EP2 // NW
NCHUNK = EPT // CH
ETP = 200192
TPT = ETP // NW
BSN = 3136

_MESH = dict(core_axis_name="c", subcore_axis_name="s", num_cores=NC,
             num_subcores=NS)


def _wid():
    return lax.axis_index("s") * NC + lax.axis_index("c")


def _counts_body(r_hbm, c_hbm, t_hbm, ones_hbm, zero_hbm, out_hbm,
                 acc, ones_v, idxb):
    cid = lax.axis_index("c")
    sid = lax.axis_index("s")
    base = _wid() * EPT
    r0 = sid * RPT
    pltpu.sync_copy(ones_hbm, ones_v)
    for v, src in enumerate((r_hbm, c_hbm, t_hbm)):
        pltpu.sync_copy(zero_hbm.at[pl.ds(r0, RPT)], acc.at[pl.ds(r0, RPT)])
        plsc.subcore_barrier()

        def chunk(ci, _, src=src):
            e0 = base + ci * CH
            pltpu.sync_copy(src.at[pl.ds(e0, CH)], idxb)
            pltpu.sync_copy(ones_v, acc.at[idxb], add=True)
            return 0

        lax.fori_loop(0, NCHUNK, chunk, 0)
        plsc.subcore_barrier()
        pltpu.sync_copy(acc.at[pl.ds(r0, RPT)],
                        out_hbm.at[cid, v, pl.ds(r0, RPT)])


def _counts(row, col, trow, ones16, zeros16):
    f = pl.kernel(
        _counts_body,
        out_type=jax.ShapeDtypeStruct((NC, 3, NPAD, 16), F32),
        mesh=plsc.VectorSubcoreMesh(**_MESH),
        compiler_params=pltpu.CompilerParams(use_tc_tiling_on_sc=False),
        scratch_types=[
            pltpu.VMEM_SHARED((NPAD, 16), F32),
            pltpu.VMEM((CH, 16), F32),
            pltpu.VMEM((CH,), jnp.int32),
        ],
    )
    return f(row, col, trow, ones16, zeros16)


def _segsum_body(x_hbm, idx_hbm, zero_hbm, out_hbm, acc, idxb, rows):
    cid = lax.axis_index("c")
    sid = lax.axis_index("s")
    base = _wid() * EPT
    r0 = sid * RPT
    pltpu.sync_copy(zero_hbm.at[pl.ds(r0, RPT)], acc.at[pl.ds(r0, RPT)])
    plsc.subcore_barrier()

    def chunk(ci, _):
        e0 = base + ci * CH
        pltpu.sync_copy(idx_hbm.at[pl.ds(e0, CH)], idxb)
        pltpu.sync_copy(x_hbm.at[pl.ds(e0, CH)], rows)
        pltpu.sync_copy(rows, acc.at[idxb], add=True)
        return 0

    lax.fori_loop(0, NCHUNK, chunk, 0)
    plsc.subcore_barrier()
    pltpu.sync_copy(acc.at[pl.ds(r0, RPT)], out_hbm.at[cid, pl.ds(r0, RPT)])


def _segsum(x, idx, zeros32):
    f = pl.kernel(
        _segsum_body,
        out_type=jax.ShapeDtypeStruct((NC, NPAD, WIDTH), F32),
        mesh=plsc.VectorSubcoreMesh(**_MESH),
        compiler_params=pltpu.CompilerParams(use_tc_tiling_on_sc=False),
        scratch_types=[
            pltpu.VMEM_SHARED((NPAD, WIDTH), F32),
            pltpu.VMEM((CH,), jnp.int32),
            pltpu.VMEM((CH, WIDTH), F32),
        ],
    )
    return f(x, idx, zeros32)


def _bcast_body(a_hbm, b_hbm, r_hbm, c_hbm, out_hbm,
                ridx, cidx, abuf, bbuf, obuf, sem1, sem2):
    base = _wid() * EPT

    def chunk(ci, _):
        e0 = base + ci * CH
        pltpu.sync_copy(r_hbm.at[pl.ds(e0, CH)], ridx)
        pltpu.sync_copy(c_hbm.at[pl.ds(e0, CH)], cidx)
        d1 = pltpu.async_copy(a_hbm.at[ridx], abuf, sem1)
        d2 = pltpu.async_copy(b_hbm.at[cidx], bbuf, sem2)
        d1.wait()
        d2.wait()

        def rowfn(r, _):
            for rr in range(4):
                for h in range(2):
                    va = abuf[r * 4 + rr, pl.ds(h * 16, 16)]
                    vb = bbuf[r * 4 + rr, pl.ds(h * 16, 16)]
                    obuf[r * 4 + rr, pl.ds(h * 16, 16)] = (
                        jnp.maximum(va + vb, 0.0))
            return 0

        lax.fori_loop(0, CH // 4, rowfn, 0)
        pltpu.sync_copy(obuf, out_hbm.at[pl.ds(e0, CH)])
        return 0

    lax.fori_loop(0, NCHUNK, chunk, 0)


def _bcast(a, b, row, col):
    f = pl.kernel(
        _bcast_body,
        out_type=jax.ShapeDtypeStruct((EP2, WIDTH), F32),
        mesh=plsc.VectorSubcoreMesh(**_MESH),
        compiler_params=pltpu.CompilerParams(use_tc_tiling_on_sc=False),
        scratch_types=[
            pltpu.VMEM((CH,), jnp.int32),
            pltpu.VMEM((CH,), jnp.int32),
            pltpu.VMEM((CH, WIDTH), F32),
            pltpu.VMEM((CH, WIDTH), F32),
            pltpu.VMEM((CH, WIDTH), F32),
            pltpu.SemaphoreType.DMA,
            pltpu.SemaphoreType.DMA,
        ],
    )
    return f(a, b, row, col)


def _final_body(u_hbm, v_hbm, tr_hbm, tc_hbm, out_hbm, ub, vb, trb, tcb, ob):
    base = _wid() * TPT
    pltpu.sync_copy(u_hbm, ub)
    pltpu.sync_copy(v_hbm, vb)
    pltpu.sync_copy(tr_hbm.at[pl.ds(base, TPT)], trb)
    pltpu.sync_copy(tc_hbm.at[pl.ds(base, TPT)], tcb)

    def g(k, _):
        ti = trb[pl.ds(k * 16, 16)]
        ci = tcb[pl.ds(k * 16, 16)]
        uv = plsc.load_gather(ub, [ti])
        vv = plsc.load_gather(vb, [ci])
        ob[pl.ds(k * 16, 16)] = uv + vv
        return 0

    lax.fori_loop(0, TPT // 16, g, 0)
    pltpu.sync_copy(ob, out_hbm.at[pl.ds(base, TPT)])


def _final(u, v, trp, tcp):
    f = pl.kernel(
        _final_body,
        out_type=jax.ShapeDtypeStruct((ETP,), F32),
        mesh=plsc.VectorSubcoreMesh(**_MESH),
        compiler_params=pltpu.CompilerParams(use_tc_tiling_on_sc=False,
                                             needs_layout_passes=False),
        scratch_types=[
            pltpu.VMEM((NPAD,), F32),
            pltpu.VMEM((NPAD,), F32),
            pltpu.VMEM((TPT,), jnp.int32),
            pltpu.VMEM((TPT,), jnp.int32),
            pltpu.VMEM((TPT,), F32),
        ],
    )
    return f(u, v, trp, tcp)


def _mlp_body(d_ref, w_ref, b_ref, o_ref):
    x = jnp.dot(d_ref[...], w_ref[...], preferred_element_type=F32)
    o_ref[...] = jnp.maximum(x + b_ref[...], 0.0)


def _mlp(datap, w_in, b_in):
    bs = 4096
    return pl.pallas_call(
        _mlp_body,
        grid=(EP2 // bs,),
        in_specs=[
            pl.BlockSpec((bs, CIN), lambda i: (i, 0)),
            pl.BlockSpec((CIN, WIDTH), lambda i: (0, 0)),
            pl.BlockSpec((1, WIDTH), lambda i: (0, 0)),
        ],
        out_specs=pl.BlockSpec((bs, WIDTH), lambda i: (i, 0)),
        out_shape=jax.ShapeDtypeStruct((EP2, WIDTH), F32),
    )(datap, w_in, b_in.reshape(1, WIDTH))


def _dense_a_body(sr, sc, st, cnt, wr, wc, wt, bp, emb_out, stats_out, accum):
    i = pl.program_id(0)
    c = jnp.clip(cnt[0, :, :, 0] + cnt[1, :, :, 0], 1.0, None)
    mr = (sr[0] + sr[1]) / c[0][:, None]
    mc = (sc[0] + sc[1]) / c[1][:, None]
    mt = (st[0] + st[1]) / c[2][:, None]
    e = (jnp.dot(mr, wr[...], preferred_element_type=F32)
         + jnp.dot(mc, wc[...], preferred_element_type=F32)
         + jnp.dot(mt, wt[...], preferred_element_type=F32) + bp[...])
    e = jnp.maximum(e, 0.0)
    emb_out[...] = e
    rid = i * BSN + lax.broadcasted_iota(jnp.int32, (BSN, 1), 0)
    m = (rid < N).astype(F32)
    em = e * m
    s1 = jnp.sum(em, axis=0, keepdims=True)
    s2 = jnp.sum(em * em, axis=0, keepdims=True)

    @pl.when(i == 0)
    def _():
        accum[...] = jnp.zeros_like(accum)

    accum[0:1, :] += s1
    accum[1:2, :] += s2

    @pl.when(i == pl.num_programs(0) - 1)
    def _():
        stats_out[...] = accum[...]


def _dense_a(sr, sc, st, cnt, wr, wc, wt, bp):
    grid = NPAD // BSN
    sspec = pl.BlockSpec((NC, BSN, WIDTH), lambda i: (0, i, 0))
    wspec = pl.BlockSpec((WIDTH, EMB), lambda i: (0, 0))
    return pl.pallas_call(
        _dense_a_body,
        grid=(grid,),
        in_specs=[
            sspec, sspec, sspec,
            pl.BlockSpec((NC, 3, BSN, 16), lambda i: (0, 0, i, 0)),
            wspec, wspec, wspec,
            pl.BlockSpec((1, EMB), lambda i: (0, 0)),
        ],
        out_specs=[
            pl.BlockSpec((BSN, EMB), lambda i: (i, 0)),
            pl.BlockSpec((2, EMB), lambda i: (0, 0)),
        ],
        out_shape=[
            jax.ShapeDtypeStruct((NPAD, EMB), F32),
            jax.ShapeDtypeStruct((2, EMB), F32),
        ],
        scratch_shapes=[pltpu.VMEM((2, EMB), F32)],
    )(sr, sc, st, cnt, wr, wc, wt, bp.reshape(1, EMB))


def _dense_b_body(emb_pre, stats, wbr, wbc, bb, a_out, b_out):
    mu = stats[0:1, :] / float(N)
    var = stats[1:2, :] / float(N) - mu * mu
    inv = lax.rsqrt(var + EPS)
    e = (emb_pre[...] - mu) * inv
    a_out[...] = jnp.dot(e, wbr[...], preferred_element_type=F32)
    b_out[...] = jnp.dot(e, wbc[...], preferred_element_type=F32) + bb[...]


def _dense_b(emb_pre, stats, wbr, wbc, bb):
    grid = NPAD // BSN
    wspec = pl.BlockSpec((EMB, WIDTH), lambda i: (0, 0))
    ospec = pl.BlockSpec((BSN, WIDTH), lambda i: (i, 0))
    return pl.pallas_call(
        _dense_b_body,
        grid=(grid,),
        in_specs=[
            pl.BlockSpec((BSN, EMB), lambda i: (i, 0)),
            pl.BlockSpec((2, EMB), lambda i: (0, 0)),
            wspec, wspec,
            pl.BlockSpec((1, WIDTH), lambda i: (0, 0)),
        ],
        out_specs=[ospec, ospec],
        out_shape=[
            jax.ShapeDtypeStruct((NPAD, WIDTH), F32),
            jax.ShapeDtypeStruct((NPAD, WIDTH), F32),
        ],
    )(emb_pre, stats, wbr, wbc, bb.reshape(1, WIDTH))


def _dense_b2_body(emb_pre, stats, wbr, wbc, bb, wout, bout, u_out, v_out):
    mu = stats[0:1, :] / float(N)
    var = stats[1:2, :] / float(N) - mu * mu
    inv = lax.rsqrt(var + EPS)
    e = (emb_pre[...] - mu) * inv
    tu = jnp.dot(wbr[...], wout[...], preferred_element_type=F32)
    tv = jnp.dot(wbc[...], wout[...], preferred_element_type=F32)
    cc = jnp.dot(bb[...], wout[...], preferred_element_type=F32) + bout[...]
    u_out[...] = jnp.dot(e, tu, preferred_element_type=F32) + cc
    v_out[...] = jnp.dot(e, tv, preferred_element_type=F32)


def _dense_b2(emb_pre, stats, wbr, wbc, bb, wout, bout):
    grid = NPAD // BSN
    wspec = pl.BlockSpec((EMB, WIDTH), lambda i: (0, 0))
    ospec = pl.BlockSpec((BSN, 1), lambda i: (i, 0))
    return pl.pallas_call(
        _dense_b2_body,
        grid=(grid,),
        in_specs=[
            pl.BlockSpec((BSN, EMB), lambda i: (i, 0)),
            pl.BlockSpec((2, EMB), lambda i: (0, 0)),
            wspec, wspec,
            pl.BlockSpec((1, WIDTH), lambda i: (0, 0)),
            pl.BlockSpec((WIDTH, 1), lambda i: (0, 0)),
            pl.BlockSpec((1, 1), lambda i: (0, 0)),
        ],
        out_specs=[ospec, ospec],
        out_shape=[
            jax.ShapeDtypeStruct((NPAD, 1), F32),
            jax.ShapeDtypeStruct((NPAD, 1), F32),
        ],
    )(emb_pre, stats, wbr, wbc, bb.reshape(1, WIDTH), wout,
      bout.reshape(1, 1))


def kernel(data, indices_identity, indices_transpose, data_embedding,
           data_target, W_in, b_in, Wpr0, Wpc0, Wpt0, bp0, Wbr0, Wbc0, bb0,
           Wpr1, Wpc1, Wpt1, bp1, Wbr1, Wbc1, bb1, W_out, b_out):
    pad_e = EP2 - E
    dead = jnp.full((pad_e,), NPAD - 1, jnp.int32)
    row = jnp.concatenate([indices_identity[0], dead])
    col = jnp.concatenate([indices_identity[1], dead])
    trow = jnp.concatenate([indices_transpose[0], dead])
    datap = jnp.concatenate([data, jnp.zeros((pad_e, CIN), F32)])
    zeros16 = jnp.zeros((NPAD, 16), F32)
    zeros32 = jnp.zeros((NPAD, WIDTH), F32)
    ones16 = jnp.ones((CH, 16), F32)

    x0 = _mlp(datap, W_in, b_in)
    cnt = _counts(row, col, trow, ones16, zeros16)

    sr0 = _segsum(x0, row, zeros32)
    sc0 = _segsum(x0, col, zeros32)
    st0 = _segsum(x0, trow, zeros32)
    emb_pre0, stats0 = _dense_a(sr0, sc0, st0, cnt, Wpr0, Wpc0, Wpt0, bp0)
    a0, b0 = _dense_b(emb_pre0, stats0, Wbr0, Wbc0, bb0)

    x1 = _bcast(a0, b0, row, col)

    sr1 = _segsum(x1, row, zeros32)
    sc1 = _segsum(x1, col, zeros32)
    st1 = _segsum(x1, trow, zeros32)
    emb_pre1, stats1 = _dense_a(sr1, sc1, st1, cnt, Wpr1, Wpc1, Wpt1, bp1)
    u, v = _dense_b2(emb_pre1, stats1, Wbr1, Wbc1, bb1, W_out, b_out)

    pad_t = ETP - ET
    zpad = jnp.zeros((pad_t,), jnp.int32)
    trp = jnp.concatenate([data_target[0], zpad])
    tcp = jnp.concatenate([data_target[1], zpad])
    outp = _final(u.reshape(NPAD), v.reshape(NPAD), trp, tcp)
    return outp[:ET].reshape(ET, 1)


# speedup vs baseline: 6.3406x; 1.5337x over previous
import functools

import jax
import jax.numpy as jnp
from jax import lax
from jax.experimental import pallas as pl
from jax.experimental.pallas import tpu as pltpu
from jax.experimental.pallas import tpu_sc as plsc

N = 50000
E = 800000
ET = 200000
CIN = 16
WIDTH = 32
EMB = 64
EPS = 1e-5
F32 = jnp.float32

NC = 2
NS = 16
NW = NC * NS
NPAD = 50176
RPT = NPAD // NS
CH = 128
EP2 = 819200
EPT = EP2 // NW
NCHUNK = EPT // CH
ETP = 200192
TPT = ETP // NW
BSN = 3136
PG = 40
NPG = NCHUNK // PG

_MESH = dict(core_axis_name="c", subcore_axis_name="s", num_cores=NC,
             num_subcores=NS)


def _wid():
    return lax.axis_index("s") * NC + lax.axis_index("c")


def _counts_body(r_hbm, c_hbm, t_hbm, ones_hbm, zero_hbm, out_hbm,
                 acc, ones_v, ib, sem):
    cid = lax.axis_index("c")
    sid = lax.axis_index("s")
    wid = _wid()
    r0 = sid * RPT
    pltpu.sync_copy(ones_hbm, ones_v)
    for v, src in enumerate((r_hbm, c_hbm, t_hbm)):
        pltpu.sync_copy(zero_hbm.at[pl.ds(r0, RPT)], acc.at[pl.ds(r0, RPT)])
        plsc.subcore_barrier()

        def page(p, _, src=src):
            pltpu.sync_copy(src.at[wid, pl.ds(p * PG, PG)], ib)

            def chunk(k, _):
                descs = [pltpu.async_copy(ones_v, acc.at[ib.at[k * 8 + j]],
                                          sem, add=True) for j in range(8)]
                for d in descs:
                    d.wait()
                return 0

            lax.fori_loop(0, PG // 8, chunk, 0)
            return 0

        lax.fori_loop(0, NPG, page, 0)
        plsc.subcore_barrier()
        pltpu.sync_copy(acc.at[pl.ds(r0, RPT)],
                        out_hbm.at[cid, v, pl.ds(r0, RPT)])


def _counts(row, col, trow, ones16, zeros16):
    f = pl.kernel(
        _counts_body,
        out_type=jax.ShapeDtypeStruct((NC, 3, NPAD, 16), F32),
        mesh=plsc.VectorSubcoreMesh(**_MESH),
        compiler_params=pltpu.CompilerParams(use_tc_tiling_on_sc=False),
        scratch_types=[
            pltpu.VMEM_SHARED((NPAD, 16), F32),
            pltpu.VMEM((CH, 16), F32),
            pltpu.VMEM((PG, CH), jnp.int32),
            pltpu.SemaphoreType.DMA,
        ],
    )
    return f(row.reshape(NW, NCHUNK, CH), col.reshape(NW, NCHUNK, CH),
             trow.reshape(NW, NCHUNK, CH), ones16, zeros16)


def _segsum_body(x_hbm, idx_hbm, zero_hbm, out_hbm,
                 acc, idxb, rows_a, rows_b, sem_a, sem_b):
    cid = lax.axis_index("c")
    sid = lax.axis_index("s")
    wid = _wid()
    base = wid * EPT
    r0 = sid * RPT
    pltpu.sync_copy(zero_hbm.at[pl.ds(r0, RPT)], acc.at[pl.ds(r0, RPT)])
    plsc.subcore_barrier()

    def ld(c, buf, sem):
        return pltpu.async_copy(x_hbm.at[pl.ds(base + c * CH, CH)], buf, sem)

    def wt(buf, sem):
        pltpu.make_async_copy(x_hbm.at[pl.ds(base, CH)], buf, sem).wait()

    ld(0, rows_a, sem_a)

    def page(p, _):
        pltpu.sync_copy(idx_hbm.at[wid, pl.ds(p * PG, PG)], idxb)

        def chunk(k, _):
            c0 = p * PG + 2 * k
            j0 = 2 * k
            ld(c0 + 1, rows_b, sem_b)
            wt(rows_a, sem_a)
            pltpu.sync_copy(rows_a, acc.at[idxb.at[j0]], add=True)

            @pl.when(c0 + 2 < NCHUNK)
            def _():
                ld(c0 + 2, rows_a, sem_a)

            wt(rows_b, sem_b)
            pltpu.sync_copy(rows_b, acc.at[idxb.at[j0 + 1]], add=True)
            return 0

        lax.fori_loop(0, PG // 2, chunk, 0)
        return 0

    lax.fori_loop(0, NPG, page, 0)
    plsc.subcore_barrier()
    pltpu.sync_copy(acc.at[pl.ds(r0, RPT)], out_hbm.at[cid, pl.ds(r0, RPT)])


def _segsum(x, idx3, zeros32):
    f = pl.kernel(
        _segsum_body,
        out_type=jax.ShapeDtypeStruct((NC, NPAD, WIDTH), F32),
        mesh=plsc.VectorSubcoreMesh(**_MESH),
        compiler_params=pltpu.CompilerParams(use_tc_tiling_on_sc=False),
        scratch_types=[
            pltpu.VMEM_SHARED((NPAD, WIDTH), F32),
            pltpu.VMEM((PG, CH), jnp.int32),
            pltpu.VMEM((CH, WIDTH), F32),
            pltpu.VMEM((CH, WIDTH), F32),
            pltpu.SemaphoreType.DMA,
            pltpu.SemaphoreType.DMA,
        ],
    )
    return f(x, idx3, zeros32)


def _bcast_body(a_hbm, b_hbm, r_hbm, c_hbm, out_hbm,
                ridxb, cidxb, a_a, b_a, o_a, a_b, b_b, o_b, sem_a, sem_b):
    wid = _wid()
    base = wid * EPT
    pltpu.sync_copy(r_hbm.at[wid], ridxb)
    pltpu.sync_copy(c_hbm.at[wid], cidxb)

    def fire(c, ab, bb, sem):
        pltpu.async_copy(a_hbm.at[ridxb.at[c]], ab, sem)
        pltpu.async_copy(b_hbm.at[cidxb.at[c]], bb, sem)

    def drain(c, ab, bb, sem):
        pltpu.make_async_copy(a_hbm.at[ridxb.at[c]], ab, sem).wait()
        pltpu.make_async_copy(b_hbm.at[cidxb.at[c]], bb, sem).wait()

    def compute(ab, bb, ob):
        def rowfn(r, _):
            for rr in range(4):
                for h in range(2):
                    va = ab[r * 4 + rr, pl.ds(h * 16, 16)]
                    vb = bb[r * 4 + rr, pl.ds(h * 16, 16)]
                    ob[r * 4 + rr, pl.ds(h * 16, 16)] = (
                        jnp.maximum(va + vb, 0.0))
            return 0

        lax.fori_loop(0, CH // 4, rowfn, 0)

    fire(0, a_a, b_a, sem_a)

    def chunk(k, _):
        c0 = 2 * k
        fire(c0 + 1, a_b, b_b, sem_b)
        drain(c0, a_a, b_a, sem_a)
        compute(a_a, b_a, o_a)
        pltpu.sync_copy(o_a, out_hbm.at[pl.ds(base + c0 * CH, CH)])

        @pl.when(c0 + 2 < NCHUNK)
        def _():
            fire(c0 + 2, a_a, b_a, sem_a)

        drain(c0 + 1, a_b, b_b, sem_b)
        compute(a_b, b_b, o_b)
        pltpu.sync_copy(o_b, out_hbm.at[pl.ds(base + (c0 + 1) * CH, CH)])
        return 0

    lax.fori_loop(0, NCHUNK // 2, chunk, 0)


def _bcast(a, b, row3, col3):
    f = pl.kernel(
        _bcast_body,
        out_type=jax.ShapeDtypeStruct((EP2, WIDTH), F32),
        mesh=plsc.VectorSubcoreMesh(**_MESH),
        compiler_params=pltpu.CompilerParams(use_tc_tiling_on_sc=False),
        scratch_types=[
            pltpu.VMEM((NCHUNK, CH), jnp.int32),
            pltpu.VMEM((NCHUNK, CH), jnp.int32),
            pltpu.VMEM((CH, WIDTH), F32),
            pltpu.VMEM((CH, WIDTH), F32),
            pltpu.VMEM((CH, WIDTH), F32),
            pltpu.VMEM((CH, WIDTH), F32),
            pltpu.VMEM((CH, WIDTH), F32),
            pltpu.VMEM((CH, WIDTH), F32),
            pltpu.SemaphoreType.DMA,
            pltpu.SemaphoreType.DMA,
        ],
    )
    return f(a, b, row3, col3)


def _final_body(u_hbm, v_hbm, tr_hbm, tc_hbm, out_hbm, ub, vb, trb, tcb, ob):
    base = _wid() * TPT
    pltpu.sync_copy(u_hbm, ub)
    pltpu.sync_copy(v_hbm, vb)
    pltpu.sync_copy(tr_hbm.at[pl.ds(base, TPT)], trb)
    pltpu.sync_copy(tc_hbm.at[pl.ds(base, TPT)], tcb)

    def g(k, _):
        ti = trb[pl.ds(k * 16, 16)]
        ci = tcb[pl.ds(k * 16, 16)]
        uv = plsc.load_gather(ub, [ti])
        vv = plsc.load_gather(vb, [ci])
        ob[pl.ds(k * 16, 16)] = uv + vv
        return 0

    lax.fori_loop(0, TPT // 16, g, 0)
    pltpu.sync_copy(ob, out_hbm.at[pl.ds(base, TPT)])


def _final(u, v, trp, tcp):
    f = pl.kernel(
        _final_body,
        out_type=jax.ShapeDtypeStruct((ETP,), F32),
        mesh=plsc.VectorSubcoreMesh(**_MESH),
        compiler_params=pltpu.CompilerParams(use_tc_tiling_on_sc=False,
                                             needs_layout_passes=False),
        scratch_types=[
            pltpu.VMEM((NPAD,), F32),
            pltpu.VMEM((NPAD,), F32),
            pltpu.VMEM((TPT,), jnp.int32),
            pltpu.VMEM((TPT,), jnp.int32),
            pltpu.VMEM((TPT,), F32),
        ],
    )
    return f(u, v, trp, tcp)


def _mlp_body(d_ref, w_ref, b_ref, o_ref):
    x = jnp.dot(d_ref[...], w_ref[...], preferred_element_type=F32)
    o_ref[...] = jnp.maximum(x + b_ref[...], 0.0)


def _mlp(datap, w_in, b_in):
    bs = 4096
    return pl.pallas_call(
        _mlp_body,
        grid=(EP2 // bs,),
        in_specs=[
            pl.BlockSpec((bs, CIN), lambda i: (i, 0)),
            pl.BlockSpec((CIN, WIDTH), lambda i: (0, 0)),
            pl.BlockSpec((1, WIDTH), lambda i: (0, 0)),
        ],
        out_specs=pl.BlockSpec((bs, WIDTH), lambda i: (i, 0)),
        out_shape=jax.ShapeDtypeStruct((EP2, WIDTH), F32),
    )(datap, w_in, b_in.reshape(1, WIDTH))


def _dense_a_body(sr, sc, st, cnt, wr, wc, wt, bp, emb_out, stats_out, accum):
    i = pl.program_id(0)
    c = jnp.clip(cnt[0, :, :, 0] + cnt[1, :, :, 0], 1.0, None)
    mr = (sr[0] + sr[1]) / c[0][:, None]
    mc = (sc[0] + sc[1]) / c[1][:, None]
    mt = (st[0] + st[1]) / c[2][:, None]
    e = (jnp.dot(mr, wr[...], preferred_element_type=F32)
         + jnp.dot(mc, wc[...], preferred_element_type=F32)
         + jnp.dot(mt, wt[...], preferred_element_type=F32) + bp[...])
    e = jnp.maximum(e, 0.0)
    emb_out[...] = e
    rid = i * BSN + lax.broadcasted_iota(jnp.int32, (BSN, 1), 0)
    m = (rid < N).astype(F32)
    em = e * m
    s1 = jnp.sum(em, axis=0, keepdims=True)
    s2 = jnp.sum(em * em, axis=0, keepdims=True)

    @pl.when(i == 0)
    def _():
        accum[...] = jnp.zeros_like(accum)

    accum[0:1, :] += s1
    accum[1:2, :] += s2

    @pl.when(i == pl.num_programs(0) - 1)
    def _():
        stats_out[...] = accum[...]


def _dense_a(sr, sc, st, cnt, wr, wc, wt, bp):
    grid = NPAD // BSN
    sspec = pl.BlockSpec((NC, BSN, WIDTH), lambda i: (0, i, 0))
    wspec = pl.BlockSpec((WIDTH, EMB), lambda i: (0, 0))
    return pl.pallas_call(
        _dense_a_body,
        grid=(grid,),
        in_specs=[
            sspec, sspec, sspec,
            pl.BlockSpec((NC, 3, BSN, 16), lambda i: (0, 0, i, 0)),
            wspec, wspec, wspec,
            pl.BlockSpec((1, EMB), lambda i: (0, 0)),
        ],
        out_specs=[
            pl.BlockSpec((BSN, EMB), lambda i: (i, 0)),
            pl.BlockSpec((2, EMB), lambda i: (0, 0)),
        ],
        out_shape=[
            jax.ShapeDtypeStruct((NPAD, EMB), F32),
            jax.ShapeDtypeStruct((2, EMB), F32),
        ],
        scratch_shapes=[pltpu.VMEM((2, EMB), F32)],
    )(sr, sc, st, cnt, wr, wc, wt, bp.reshape(1, EMB))


def _dense_b_body(emb_pre, stats, wbr, wbc, bb, a_out, b_out):
    mu = stats[0:1, :] / float(N)
    var = stats[1:2, :] / float(N) - mu * mu
    inv = lax.rsqrt(var + EPS)
    e = (emb_pre[...] - mu) * inv
    a_out[...] = jnp.dot(e, wbr[...], preferred_element_type=F32)
    b_out[...] = jnp.dot(e, wbc[...], preferred_element_type=F32) + bb[...]


def _dense_b(emb_pre, stats, wbr, wbc, bb):
    grid = NPAD // BSN
    wspec = pl.BlockSpec((EMB, WIDTH), lambda i: (0, 0))
    ospec = pl.BlockSpec((BSN, WIDTH), lambda i: (i, 0))
    return pl.pallas_call(
        _dense_b_body,
        grid=(grid,),
        in_specs=[
            pl.BlockSpec((BSN, EMB), lambda i: (i, 0)),
            pl.BlockSpec((2, EMB), lambda i: (0, 0)),
            wspec, wspec,
            pl.BlockSpec((1, WIDTH), lambda i: (0, 0)),
        ],
        out_specs=[ospec, ospec],
        out_shape=[
            jax.ShapeDtypeStruct((NPAD, WIDTH), F32),
            jax.ShapeDtypeStruct((NPAD, WIDTH), F32),
        ],
    )(emb_pre, stats, wbr, wbc, bb.reshape(1, WIDTH))


def _dense_b2_body(emb_pre, stats, wbr, wbc, bb, wout, bout, u_out, v_out):
    mu = stats[0:1, :] / float(N)
    var = stats[1:2, :] / float(N) - mu * mu
    inv = lax.rsqrt(var + EPS)
    e = (emb_pre[...] - mu) * inv
    tu = jnp.dot(wbr[...], wout[...], preferred_element_type=F32)
    tv = jnp.dot(wbc[...], wout[...], preferred_element_type=F32)
    cc = jnp.dot(bb[...], wout[...], preferred_element_type=F32) + bout[...]
    u_out[...] = jnp.dot(e, tu, preferred_element_type=F32) + cc
    v_out[...] = jnp.dot(e, tv, preferred_element_type=F32)


def _dense_b2(emb_pre, stats, wbr, wbc, bb, wout, bout):
    grid = NPAD // BSN
    wspec = pl.BlockSpec((EMB, WIDTH), lambda i: (0, 0))
    ospec = pl.BlockSpec((BSN, 1), lambda i: (i, 0))
    return pl.pallas_call(
        _dense_b2_body,
        grid=(grid,),
        in_specs=[
            pl.BlockSpec((BSN, EMB), lambda i: (i, 0)),
            pl.BlockSpec((2, EMB), lambda i: (0, 0)),
            wspec, wspec,
            pl.BlockSpec((1, WIDTH), lambda i: (0, 0)),
            pl.BlockSpec((WIDTH, 1), lambda i: (0, 0)),
            pl.BlockSpec((1, 1), lambda i: (0, 0)),
        ],
        out_specs=[ospec, ospec],
        out_shape=[
            jax.ShapeDtypeStruct((NPAD, 1), F32),
            jax.ShapeDtypeStruct((NPAD, 1), F32),
        ],
    )(emb_pre, stats, wbr, wbc, bb.reshape(1, WIDTH), wout,
      bout.reshape(1, 1))


def kernel(data, indices_identity, indices_transpose, data_embedding,
           data_target, W_in, b_in, Wpr0, Wpc0, Wpt0, bp0, Wbr0, Wbc0, bb0,
           Wpr1, Wpc1, Wpt1, bp1, Wbr1, Wbc1, bb1, W_out, b_out):
    pad_e = EP2 - E
    dead = jnp.full((pad_e,), NPAD - 1, jnp.int32)
    row = jnp.concatenate([indices_identity[0], dead])
    col = jnp.concatenate([indices_identity[1], dead])
    trow = jnp.concatenate([indices_transpose[0], dead])
    datap = jnp.concatenate([data, jnp.zeros((pad_e, CIN), F32)])
    zeros16 = jnp.zeros((NPAD, 16), F32)
    zeros32 = jnp.zeros((NPAD, WIDTH), F32)
    ones16 = jnp.ones((CH, 16), F32)

    row3 = row.reshape(NW, NCHUNK, CH)
    col3 = col.reshape(NW, NCHUNK, CH)
    trow3 = trow.reshape(NW, NCHUNK, CH)

    x0 = _mlp(datap, W_in, b_in)
    cnt = _counts(row, col, trow, ones16, zeros16)

    sr0 = _segsum(x0, row3, zeros32)
    sc0 = _segsum(x0, col3, zeros32)
    st0 = _segsum(x0, trow3, zeros32)
    emb_pre0, stats0 = _dense_a(sr0, sc0, st0, cnt, Wpr0, Wpc0, Wpt0, bp0)
    a0, b0 = _dense_b(emb_pre0, stats0, Wbr0, Wbc0, bb0)

    x1 = _bcast(a0, b0, row3, col3)

    sr1 = _segsum(x1, row3, zeros32)
    sc1 = _segsum(x1, col3, zeros32)
    st1 = _segsum(x1, trow3, zeros32)
    emb_pre1, stats1 = _dense_a(sr1, sc1, st1, cnt, Wpr1, Wpc1, Wpt1, bp1)
    u, v = _dense_b2(emb_pre1, stats1, Wbr1, Wbc1, bb1, W_out, b_out)

    pad_t = ETP - ET
    zpad = jnp.zeros((pad_t,), jnp.int32)
    trp = jnp.concatenate([data_target[0], zpad])
    tcp = jnp.concatenate([data_target[1], zpad])
    outp = _final(u.reshape(NPAD), v.reshape(NPAD), trp, tcp)
    return outp[:ET].reshape(ET, 1)
